# baseline (speedup 1.0000x reference)
import functools

import jax
import jax.numpy as jnp
from jax.experimental import pallas as pl
from jax.experimental.pallas import tpu as pltpu

_OVERLAP_THRESHOLD = 0.3
_N_MAX_OBJECTS = 20
_LANE = 128


def _nms_body(n_real, n_pad, p_ref, bx_ref, by_ref, bw_ref, bh_ref,
              op_ref, ox_ref, oy_ref, ow_ref, oh_ref, s_ref, e_ref):
    p = p_ref[...]
    bx = bx_ref[...]
    by = by_ref[...]
    bw = bw_ref[...]
    bh = bh_ref[...]
    r, h = p.shape
    b = r // 2

    x1 = bx - 0.5 * bw
    x3 = bx + 0.5 * bw
    y1 = by - 0.5 * bh
    y3 = by + 0.5 * bh
    area = bw * bh

    sub = jax.lax.broadcasted_iota(jnp.int32, (r, h), 0)
    col = (jax.lax.broadcasted_iota(jnp.int32, (r, h), 1)
           + jnp.where(sub >= b, h, 0))
    slot = jax.lax.broadcasted_iota(jnp.int32, (b, _LANE), 1)

    def pmerge(v):
        return jnp.concatenate([v[b:], v[:b]], axis=0)

    s_ref[...] = p
    e_ref[...] = jnp.where(col >= n_real, 1.0, 0.0)
    zacc = jnp.zeros((b, _LANE), dtype=jnp.float32)
    op_ref[...] = zacc
    ox_ref[...] = zacc
    oy_ref[...] = zacc
    ow_ref[...] = zacc
    oh_ref[...] = zacc

    def body(l, _):
        s = s_ref[...]
        r1 = jnp.max(s, axis=1, keepdims=True)
        pmax = jnp.maximum(r1, pmerge(r1))
        valid = pmax > 0.0
        vf = jnp.where(valid, 1.0, 0.0)
        m1 = jnp.min(jnp.where(s == pmax, col, n_pad), axis=1, keepdims=True)
        m = jnp.minimum(m1, pmerge(m1))
        sel = (col == m).astype(jnp.float32) * vf

        def pick(v):
            pk = jnp.sum(sel * v, axis=1, keepdims=True)
            return pk + pmerge(pk)

        bxm = pick(bx)
        bym = pick(by)
        bwm = pick(bw)
        bhm = pick(bh)

        x1m = bxm - 0.5 * bwm
        x3m = bxm + 0.5 * bwm
        y1m = bym - 0.5 * bhm
        y3m = bym + 0.5 * bhm
        aream = bwm * bhm
        inter = (jnp.maximum(jnp.minimum(x3, x3m) - jnp.maximum(x1, x1m), 0.0)
                 * jnp.maximum(jnp.minimum(y3, y3m) - jnp.maximum(y1, y1m), 0.0))
        ov = jnp.where(inter / jnp.minimum(area, aream) > _OVERLAP_THRESHOLD,
                       vf, 0.0)
        s_ref[...] = s * (1.0 - ov)
        e_ref[...] = jnp.maximum(e_ref[...], sel)

        at = slot == l
        val = jnp.where(valid, pmax, 0.0)
        op_ref[...] = jnp.where(at, val[:b], op_ref[...])
        ox_ref[...] = jnp.where(at, bxm[:b], ox_ref[...])
        oy_ref[...] = jnp.where(at, bym[:b], oy_ref[...])
        ow_ref[...] = jnp.where(at, bwm[:b], ow_ref[...])
        oh_ref[...] = jnp.where(at, bhm[:b], oh_ref[...])
        return 0

    jax.lax.fori_loop(0, _N_MAX_OBJECTS, body, 0)

    used = jnp.where(slot < _N_MAX_OBJECTS, op_ref[...], 1.0)
    some_empty = jnp.min(used) == 0.0

    @pl.when(some_empty)
    def _fill():
        def fbody(l, _):
            at = slot == l
            cur = jnp.sum(jnp.where(at, op_ref[...], 0.0), axis=1,
                          keepdims=True)
            empty4 = (cur == 0.0).astype(jnp.float32)
            empty = jnp.concatenate([empty4, empty4], axis=0)
            e = e_ref[...]
            m1 = jnp.min(jnp.where(e > 0.0, n_pad, col), axis=1, keepdims=True)
            m2 = jnp.minimum(m1, pmerge(m1))
            sel = (col == m2).astype(jnp.float32) * empty
            e_ref[...] = jnp.maximum(e, sel)

            def pick(v):
                pk = jnp.sum(sel * v, axis=1, keepdims=True)
                return pk + pmerge(pk)

            w = at & (empty4 > 0.0)
            ox_ref[...] = jnp.where(w, pick(bx)[:b], ox_ref[...])
            oy_ref[...] = jnp.where(w, pick(by)[:b], oy_ref[...])
            ow_ref[...] = jnp.where(w, pick(bw)[:b], ow_ref[...])
            oh_ref[...] = jnp.where(w, pick(bh)[:b], oh_ref[...])
            return 0

        jax.lax.fori_loop(0, _N_MAX_OBJECTS, fbody, 0)


@jax.jit
def kernel(prob, bx_dimfull, by_dimfull, bw_dimfull, bh_dimfull):
    b, n, _ = prob.shape
    n_pad = ((n + 2 * _LANE - 1) // (2 * _LANE)) * (2 * _LANE)
    half = n_pad // 2

    def prep(v, fill):
        v = v[..., 0]
        v = jnp.pad(v, ((0, 0), (0, n_pad - n)), constant_values=fill)
        return v.reshape(b, 2, half).transpose(1, 0, 2).reshape(2 * b, half)

    p = prep(prob, 0.0)
    bx = prep(bx_dimfull, 0.0)
    by = prep(by_dimfull, 0.0)
    bw = prep(bw_dimfull, 1.0)
    bh = prep(bh_dimfull, 1.0)

    out = jax.ShapeDtypeStruct((b, _LANE), jnp.float32)
    ap, ax, ay, aw, ah = pl.pallas_call(
        functools.partial(_nms_body, n, n_pad),
        out_shape=(out, out, out, out, out),
        scratch_shapes=[
            pltpu.VMEM((2 * b, half), jnp.float32),
            pltpu.VMEM((2 * b, half), jnp.float32),
        ],
    )(p, bx, by, bw, bh)

    k = min(_N_MAX_OBJECTS, n)
    return (ap[:, :k, None], ax[:, :k, None], ay[:, :k, None],
            aw[:, :k, None], ah[:, :k, None])


# speedup vs baseline: 1.0038x; 1.0038x over previous
import functools

import jax
import jax.numpy as jnp
from jax.experimental import pallas as pl
from jax.experimental.pallas import tpu as pltpu

_OVERLAP_THRESHOLD = 0.3
_N_MAX_OBJECTS = 20
_LANE = 128


def _nms_body(n_real, n_pad, p_ref, bx_ref, by_ref, bw_ref, bh_ref,
              op_ref, ox_ref, oy_ref, ow_ref, oh_ref, s_ref, e_ref):
    h = n_pad // 2

    def fold(ref):
        v = ref[...]
        return jnp.concatenate([v[:, :h], v[:, h:]], axis=0)

    p = fold(p_ref)
    bx = fold(bx_ref)
    by = fold(by_ref)
    bw = fold(bw_ref)
    bh = fold(bh_ref)
    r = p.shape[0]
    b = r // 2

    x1 = bx - 0.5 * bw
    x3 = bx + 0.5 * bw
    y1 = by - 0.5 * bh
    y3 = by + 0.5 * bh
    area = bw * bh

    sub = jax.lax.broadcasted_iota(jnp.int32, (r, h), 0)
    col = (jax.lax.broadcasted_iota(jnp.int32, (r, h), 1)
           + jnp.where(sub >= b, h, 0))
    slot = jax.lax.broadcasted_iota(jnp.int32, (b, _LANE), 1)

    def pmerge(v):
        return jnp.concatenate([v[b:], v[:b]], axis=0)

    s_ref[...] = p
    e_ref[...] = jnp.where(col >= n_real, 1.0, 0.0)
    zacc = jnp.zeros((b, _LANE), dtype=jnp.float32)
    op_ref[...] = zacc
    ox_ref[...] = zacc
    oy_ref[...] = zacc
    ow_ref[...] = zacc
    oh_ref[...] = zacc

    def body(l, _):
        s = s_ref[...]
        r1 = jnp.max(s, axis=1, keepdims=True)
        pmax = jnp.maximum(r1, pmerge(r1))
        valid = pmax > 0.0
        vf = jnp.where(valid, 1.0, 0.0)
        m1 = jnp.min(jnp.where(s == pmax, col, n_pad), axis=1, keepdims=True)
        m = jnp.minimum(m1, pmerge(m1))
        sel = (col == m).astype(jnp.float32) * vf

        def pick(v):
            pk = jnp.sum(sel * v, axis=1, keepdims=True)
            return pk + pmerge(pk)

        bxm = pick(bx)
        bym = pick(by)
        bwm = pick(bw)
        bhm = pick(bh)

        x1m = bxm - 0.5 * bwm
        x3m = bxm + 0.5 * bwm
        y1m = bym - 0.5 * bhm
        y3m = bym + 0.5 * bhm
        aream = bwm * bhm
        inter = (jnp.maximum(jnp.minimum(x3, x3m) - jnp.maximum(x1, x1m), 0.0)
                 * jnp.maximum(jnp.minimum(y3, y3m) - jnp.maximum(y1, y1m), 0.0))
        ov = jnp.where(inter / jnp.minimum(area, aream) > _OVERLAP_THRESHOLD,
                       vf, 0.0)
        s_ref[...] = s * (1.0 - ov)
        e_ref[...] = jnp.maximum(e_ref[...], sel)

        at = slot == l
        val = jnp.where(valid, pmax, 0.0)
        op_ref[...] = jnp.where(at, val[:b], op_ref[...])
        ox_ref[...] = jnp.where(at, bxm[:b], ox_ref[...])
        oy_ref[...] = jnp.where(at, bym[:b], oy_ref[...])
        ow_ref[...] = jnp.where(at, bwm[:b], ow_ref[...])
        oh_ref[...] = jnp.where(at, bhm[:b], oh_ref[...])
        return 0

    jax.lax.fori_loop(0, _N_MAX_OBJECTS, body, 0)

    used = jnp.where(slot < _N_MAX_OBJECTS, op_ref[...], 1.0)
    some_empty = jnp.min(used) == 0.0

    @pl.when(some_empty)
    def _fill():
        def fbody(l, _):
            at = slot == l
            cur = jnp.sum(jnp.where(at, op_ref[...], 0.0), axis=1,
                          keepdims=True)
            empty4 = (cur == 0.0).astype(jnp.float32)
            empty = jnp.concatenate([empty4, empty4], axis=0)
            e = e_ref[...]
            m1 = jnp.min(jnp.where(e > 0.0, n_pad, col), axis=1, keepdims=True)
            m2 = jnp.minimum(m1, pmerge(m1))
            sel = (col == m2).astype(jnp.float32) * empty
            e_ref[...] = jnp.maximum(e, sel)

            def pick(v):
                pk = jnp.sum(sel * v, axis=1, keepdims=True)
                return pk + pmerge(pk)

            w = at & (empty4 > 0.0)
            ox_ref[...] = jnp.where(w, pick(bx)[:b], ox_ref[...])
            oy_ref[...] = jnp.where(w, pick(by)[:b], oy_ref[...])
            ow_ref[...] = jnp.where(w, pick(bw)[:b], ow_ref[...])
            oh_ref[...] = jnp.where(w, pick(bh)[:b], oh_ref[...])
            return 0

        jax.lax.fori_loop(0, _N_MAX_OBJECTS, fbody, 0)


@jax.jit
def kernel(prob, bx_dimfull, by_dimfull, bw_dimfull, bh_dimfull):
    b, n, _ = prob.shape
    n_pad = ((n + 2 * _LANE - 1) // (2 * _LANE)) * (2 * _LANE)
    half = n_pad // 2

    def prep(v, fill):
        v = v[..., 0]
        return jnp.pad(v, ((0, 0), (0, n_pad - n)), constant_values=fill)

    p = prep(prob, 0.0)
    bx = prep(bx_dimfull, 0.0)
    by = prep(by_dimfull, 0.0)
    bw = prep(bw_dimfull, 1.0)
    bh = prep(bh_dimfull, 1.0)

    out = jax.ShapeDtypeStruct((b, _LANE), jnp.float32)
    ap, ax, ay, aw, ah = pl.pallas_call(
        functools.partial(_nms_body, n, n_pad),
        out_shape=(out, out, out, out, out),
        scratch_shapes=[
            pltpu.VMEM((2 * b, half), jnp.float32),
            pltpu.VMEM((2 * b, half), jnp.float32),
        ],
    )(p, bx, by, bw, bh)

    k = min(_N_MAX_OBJECTS, n)
    return (ap[:, :k, None], ax[:, :k, None], ay[:, :k, None],
            aw[:, :k, None], ah[:, :k, None])


# speedup vs baseline: 1.1268x; 1.1225x over previous
import functools

import jax
import jax.numpy as jnp
from jax.experimental import pallas as pl
from jax.experimental.pallas import tpu as pltpu

_P_THRESHOLD = 0.1
_OVERLAP_THRESHOLD = 0.3
_N_MAX_OBJECTS = 20
_LANE = 128


def _nms_body(n_real, p_ref, bx_ref, by_ref, bw_ref, bh_ref,
              op_ref, ox_ref, oy_ref, ow_ref, oh_ref, s_ref, e_ref):
    p = p_ref[...]
    bx = bx_ref[...]
    by = by_ref[...]
    bw = bw_ref[...]
    bh = bh_ref[...]
    b, n = p.shape

    x1 = bx - 0.5 * bw
    x3 = bx + 0.5 * bw
    y1 = by - 0.5 * bh
    y3 = by + 0.5 * bh
    area = bw * bh

    col = jax.lax.broadcasted_iota(jnp.int32, (b, n), 1)
    slot = jax.lax.broadcasted_iota(jnp.int32, (b, _LANE), 1)

    s_ref[...] = p
    e_ref[...] = jnp.where(col >= n_real, 1.0, 0.0)
    zacc = jnp.zeros((b, _LANE), dtype=jnp.float32)
    op_ref[...] = zacc
    ox_ref[...] = zacc
    oy_ref[...] = zacc
    ow_ref[...] = zacc
    oh_ref[...] = zacc

    def body(l, _):
        s = s_ref[...]
        pmax = jnp.max(s, axis=1, keepdims=True)
        valid = pmax > 0.0
        vf = jnp.where(valid, 1.0, 0.0)
        m = jnp.min(jnp.where(s == pmax, col, n), axis=1, keepdims=True)
        sel = (col == m).astype(jnp.float32) * vf

        def pick(v):
            return jnp.sum(sel * v, axis=1, keepdims=True)

        bxm = pick(bx)
        bym = pick(by)
        bwm = pick(bw)
        bhm = pick(bh)

        x1m = bxm - 0.5 * bwm
        x3m = bxm + 0.5 * bwm
        y1m = bym - 0.5 * bhm
        y3m = bym + 0.5 * bhm
        aream = bwm * bhm
        inter = (jnp.maximum(jnp.minimum(x3, x3m) - jnp.maximum(x1, x1m), 0.0)
                 * jnp.maximum(jnp.minimum(y3, y3m) - jnp.maximum(y1, y1m), 0.0))
        ov = jnp.where(inter / jnp.minimum(area, aream) > _OVERLAP_THRESHOLD,
                       vf, 0.0)
        s_ref[...] = s * (1.0 - ov)
        e_ref[...] = jnp.maximum(e_ref[...], sel)

        at = slot == l
        op_ref[...] = jnp.where(at, jnp.where(valid, pmax, 0.0), op_ref[...])
        ox_ref[...] = jnp.where(at, bxm, ox_ref[...])
        oy_ref[...] = jnp.where(at, bym, oy_ref[...])
        ow_ref[...] = jnp.where(at, bwm, ow_ref[...])
        oh_ref[...] = jnp.where(at, bhm, oh_ref[...])
        return 0

    jax.lax.fori_loop(0, _N_MAX_OBJECTS, body, 0)

    used = jnp.where(slot < _N_MAX_OBJECTS, op_ref[...], 1.0)
    some_empty = jnp.min(used) == 0.0

    @pl.when(some_empty)
    def _fill():
        def fbody(l, _):
            at = slot == l
            cur = jnp.sum(jnp.where(at, op_ref[...], 0.0), axis=1,
                          keepdims=True)
            empty = (cur == 0.0).astype(jnp.float32)
            e = e_ref[...]
            m2 = jnp.min(jnp.where(e > 0.0, n, col), axis=1, keepdims=True)
            sel = (col == m2).astype(jnp.float32) * empty
            e_ref[...] = jnp.maximum(e, sel)

            def pick(v):
                return jnp.sum(sel * v, axis=1, keepdims=True)

            w = at & (empty > 0.0)
            ox_ref[...] = jnp.where(w, pick(bx), ox_ref[...])
            oy_ref[...] = jnp.where(w, pick(by), oy_ref[...])
            ow_ref[...] = jnp.where(w, pick(bw), ow_ref[...])
            oh_ref[...] = jnp.where(w, pick(bh), oh_ref[...])
            return 0

        jax.lax.fori_loop(0, _N_MAX_OBJECTS, fbody, 0)


@jax.jit
def kernel(prob, bx_dimfull, by_dimfull, bw_dimfull, bh_dimfull):
    b, n, _ = prob.shape
    n_pad = ((n + _LANE - 1) // _LANE) * _LANE

    def prep(v, fill):
        v = v[..., 0]
        return jnp.pad(v, ((0, 0), (0, n_pad - n)), constant_values=fill)

    p = prep(prob, 0.0)
    bx = prep(bx_dimfull, 0.0)
    by = prep(by_dimfull, 0.0)
    bw = prep(bw_dimfull, 1.0)
    bh = prep(bh_dimfull, 1.0)

    out = jax.ShapeDtypeStruct((b, _LANE), jnp.float32)
    ap, ax, ay, aw, ah = pl.pallas_call(
        functools.partial(_nms_body, n),
        out_shape=(out, out, out, out, out),
        scratch_shapes=[
            pltpu.VMEM((b, n_pad), jnp.float32),
            pltpu.VMEM((b, n_pad), jnp.float32),
        ],
    )(p, bx, by, bw, bh)

    k = min(_N_MAX_OBJECTS, n)
    return (ap[:, :k, None], ax[:, :k, None], ay[:, :k, None],
            aw[:, :k, None], ah[:, :k, None])


# speedup vs baseline: 1.1572x; 1.0269x over previous
import functools

import jax
import jax.numpy as jnp
from jax.experimental import pallas as pl
from jax.experimental.pallas import tpu as pltpu

_P_THRESHOLD = 0.1
_OVERLAP_THRESHOLD = 0.3
_N_MAX_OBJECTS = 20
_LANE = 128


def _nms_body(n_real, p_ref, bx_ref, by_ref, bw_ref, bh_ref,
              op_ref, ox_ref, oy_ref, ow_ref, oh_ref, s_ref, e_ref, oi_ref):
    p = p_ref[...]
    bx = bx_ref[...]
    by = by_ref[...]
    bw = bw_ref[...]
    bh = bh_ref[...]
    b, n = p.shape

    x1 = bx - 0.5 * bw
    x3 = bx + 0.5 * bw
    y1 = by - 0.5 * bh
    y3 = by + 0.5 * bh
    area = bw * bh

    col = jax.lax.broadcasted_iota(jnp.int32, (b, n), 1)
    slot = jax.lax.broadcasted_iota(jnp.int32, (b, _LANE), 1)

    s_ref[...] = p
    oi_ref[...] = jnp.full((b, _LANE), n, jnp.int32)
    zacc = jnp.zeros((b, _LANE), dtype=jnp.float32)
    op_ref[...] = zacc
    ox_ref[...] = zacc
    oy_ref[...] = zacc
    ow_ref[...] = zacc
    oh_ref[...] = zacc

    def body(l, _):
        s = s_ref[...]
        pmax = jnp.max(s, axis=1, keepdims=True)
        valid = pmax > 0.0
        vf = jnp.where(valid, 1.0, 0.0)
        m = jnp.min(jnp.where(s == pmax, col, n), axis=1, keepdims=True)
        sel = (col == m).astype(jnp.float32)

        def pick(v):
            return jnp.sum(sel * v, axis=1, keepdims=True)

        bxm = pick(bx)
        bym = pick(by)
        bwm = pick(bw)
        bhm = pick(bh)

        x1m = bxm - 0.5 * bwm
        x3m = bxm + 0.5 * bwm
        y1m = bym - 0.5 * bhm
        y3m = bym + 0.5 * bhm
        aream = bwm * bhm
        inter = (jnp.maximum(jnp.minimum(x3, x3m) - jnp.maximum(x1, x1m), 0.0)
                 * jnp.maximum(jnp.minimum(y3, y3m) - jnp.maximum(y1, y1m), 0.0))
        ov = jnp.where(inter / jnp.minimum(area, aream) > _OVERLAP_THRESHOLD,
                       vf, 0.0)
        s_ref[...] = s * (1.0 - ov)

        at = slot == l
        oi_ref[...] = jnp.where(at & valid, m, oi_ref[...])
        op_ref[...] = jnp.where(at, jnp.where(valid, pmax, 0.0), op_ref[...])
        ox_ref[...] = jnp.where(at, bxm, ox_ref[...])
        oy_ref[...] = jnp.where(at, bym, oy_ref[...])
        ow_ref[...] = jnp.where(at, bwm, ow_ref[...])
        oh_ref[...] = jnp.where(at, bhm, oh_ref[...])
        return 0

    jax.lax.fori_loop(0, _N_MAX_OBJECTS, body, 0)

    used = jnp.where(slot < _N_MAX_OBJECTS, op_ref[...], 1.0)
    some_empty = jnp.min(used) == 0.0

    @pl.when(some_empty)
    def _fill():
        e_ref[...] = jnp.where(col >= n_real, 1.0, 0.0)

        def ebody(l, _):
            idxl = jnp.sum(jnp.where(slot == l, oi_ref[...], 0), axis=1,
                           keepdims=True)
            e_ref[...] = jnp.maximum(e_ref[...],
                                     (col == idxl).astype(jnp.float32))
            return 0

        jax.lax.fori_loop(0, _N_MAX_OBJECTS, ebody, 0)

        def fbody(l, _):
            at = slot == l
            cur = jnp.sum(jnp.where(at, op_ref[...], 0.0), axis=1,
                          keepdims=True)
            empty = (cur == 0.0).astype(jnp.float32)
            e = e_ref[...]
            m2 = jnp.min(jnp.where(e > 0.0, n, col), axis=1, keepdims=True)
            sel = (col == m2).astype(jnp.float32) * empty
            e_ref[...] = jnp.maximum(e, sel)

            def pick(v):
                return jnp.sum(sel * v, axis=1, keepdims=True)

            w = at & (empty > 0.0)
            ox_ref[...] = jnp.where(w, pick(bx), ox_ref[...])
            oy_ref[...] = jnp.where(w, pick(by), oy_ref[...])
            ow_ref[...] = jnp.where(w, pick(bw), ow_ref[...])
            oh_ref[...] = jnp.where(w, pick(bh), oh_ref[...])
            return 0

        jax.lax.fori_loop(0, _N_MAX_OBJECTS, fbody, 0)


@jax.jit
def kernel(prob, bx_dimfull, by_dimfull, bw_dimfull, bh_dimfull):
    b, n, _ = prob.shape
    n_pad = ((n + _LANE - 1) // _LANE) * _LANE

    def prep(v, fill):
        v = v[..., 0]
        return jnp.pad(v, ((0, 0), (0, n_pad - n)), constant_values=fill)

    p = prep(prob, 0.0)
    bx = prep(bx_dimfull, 0.0)
    by = prep(by_dimfull, 0.0)
    bw = prep(bw_dimfull, 1.0)
    bh = prep(bh_dimfull, 1.0)

    out = jax.ShapeDtypeStruct((b, _LANE), jnp.float32)
    ap, ax, ay, aw, ah = pl.pallas_call(
        functools.partial(_nms_body, n),
        out_shape=(out, out, out, out, out),
        scratch_shapes=[
            pltpu.VMEM((b, n_pad), jnp.float32),
            pltpu.VMEM((b, n_pad), jnp.float32),
            pltpu.VMEM((b, _LANE), jnp.int32),
        ],
    )(p, bx, by, bw, bh)

    k = min(_N_MAX_OBJECTS, n)
    return (ap[:, :k, None], ax[:, :k, None], ay[:, :k, None],
            aw[:, :k, None], ah[:, :k, None])
